# Initial kernel scaffold; baseline (speedup 1.0000x reference)
import jax
import jax.numpy as jnp
from jax.experimental import pallas as pl


def kernel(queries, items):
    raise NotImplementedError("write your pallas kernel here")


# speedup vs baseline: 247.9546x; 247.9546x over previous
import functools

import jax
import jax.numpy as jnp
from jax.experimental import pallas as pl


def _loss_kernel(q_ref, items_ref, out_ref, *, K, M, CH):
    i = pl.program_id(0)
    qb = q_ref[0]
    T = qb.shape[0]
    qnorm = jnp.sum(qb * qb, axis=1, keepdims=True)

    best_val = None
    for c in range(M // CH):
        ic = items_ref[pl.ds(c * CH, CH), :]
        s = jax.lax.dot_general(
            qb, ic, (((1,), (1,)), ((), ())),
            preferred_element_type=jnp.float32)
        norms = jnp.sum(ic * ic, axis=1)[None, :]
        gidx = jax.lax.broadcasted_iota(jnp.int32, (T, CH), 1) + c * CH
        cmax = jnp.max(s, axis=1, keepdims=True)
        mask = s == cmax
        lidx = jnp.min(jnp.where(mask, gidx, M), axis=1, keepdims=True)
        lnorm = jnp.max(
            jnp.where(mask & (gidx == lidx), norms, -jnp.inf),
            axis=1, keepdims=True)
        if best_val is None:
            best_val, best_idx, best_norm = cmax, lidx, lnorm
        else:
            better = (cmax > best_val) | ((cmax == best_val) & (lidx < best_idx))
            best_val = jnp.where(better, cmax, best_val)
            best_idx = jnp.where(better, lidx, best_idx)
            best_norm = jnp.where(better, lnorm, best_norm)

    loss = qnorm - 2.0 * best_val + best_norm
    row = jnp.reshape(loss, (T,))

    @pl.when(i % K == 0)
    def _init():
        out_ref[0, 0, :] = row

    @pl.when(i % K != 0)
    def _acc():
        out_ref[0, 0, :] = out_ref[0, 0, :] + row


def kernel(queries, items):
    B, K, T, N = queries.shape
    M = items.shape[0]
    q = queries.reshape(B * K, T, N)
    CH = 2048
    out = pl.pallas_call(
        functools.partial(_loss_kernel, K=K, M=M, CH=CH),
        grid=(B * K,),
        in_specs=[
            pl.BlockSpec((1, T, N), lambda i: (i, 0, 0)),
            pl.BlockSpec((M, N), lambda i: (0, 0)),
        ],
        out_specs=pl.BlockSpec((1, 1, T), lambda i: (i // K, 0, 0)),
        out_shape=jax.ShapeDtypeStruct((B, 1, T), jnp.float32),
    )(q, items)
    return out.reshape(B, T)
